# baseline (speedup 1.0000x reference)
import functools

import jax
import jax.numpy as jnp
from jax import lax
from jax.experimental import pallas as pl
from jax.experimental.pallas import tpu as pltpu
from jax.experimental.pallas import tpu_sc as plsc

N = 10000
E = 320000
D = 128
NC, NS = 2, 16
CH = 128
NPAD = 10240
NCHW = 79
CHS = 64
NCHS = 158
EPAD = NC * NS * NCHW * CH
RPT = NPAD // NS
BN_EPS = 1e-5

_mesh = plsc.VectorSubcoreMesh(core_axis_name="c", subcore_axis_name="s")


@functools.partial(
    pl.kernel,
    out_type=jax.ShapeDtypeStruct((NC, NPAD), jnp.float32),
    mesh=_mesh,
    scratch_types=[
        pltpu.VMEM((NCHW * CH,), jnp.int32),
        pltpu.VMEM((CH,), jnp.float32),
        pltpu.VMEM_SHARED((NPAD,), jnp.float32),
        pltpu.SemaphoreType.DMA,
    ],
)
def _hist_kernel(dst_hbm, zeros_hbm, out_hbm, dst_v, ones_v, hist_s, hsem):
    c = lax.axis_index("c")
    s = lax.axis_index("s")
    pltpu.sync_copy(zeros_hbm.at[pl.ds(s * RPT, RPT)],
                    hist_s.at[pl.ds(s * RPT, RPT)])
    pltpu.sync_copy(dst_hbm.at[c * NS + s], dst_v)
    for r in range(CH // 16):
        ones_v[pl.ds(r * 16, 16)] = jnp.ones((16,), jnp.float32)
    plsc.subcore_barrier()

    def body(j, carry):
        pltpu.async_copy(ones_v, hist_s.at[dst_v.at[pl.ds(j * CH, CH)]],
                         hsem, add=True)

        @pl.when(j >= 8)
        def _():
            pltpu.make_async_copy(ones_v, hist_s.at[dst_v.at[pl.ds(0, CH)]],
                              hsem).wait()

        return carry

    lax.fori_loop(0, NCHW, body, 0)

    def drain(j, carry):
        pltpu.make_async_copy(ones_v, hist_s.at[dst_v.at[pl.ds(0, CH)]],
                              hsem).wait()
        return carry

    lax.fori_loop(0, 8, drain, 0)
    plsc.subcore_barrier()
    pltpu.sync_copy(hist_s.at[pl.ds(s * RPT, RPT)],
                    out_hbm.at[c].at[pl.ds(s * RPT, RPT)])


@functools.partial(
    pl.kernel,
    out_type=jax.ShapeDtypeStruct((NC, NPAD, D), jnp.float32),
    mesh=_mesh,
    scratch_types=[
        pltpu.VMEM((NCHS * CHS,), jnp.int32),
        pltpu.VMEM((NCHS * CHS,), jnp.int32),
        [pltpu.VMEM((CHS // 4, D), jnp.float32)] * 12,
        pltpu.VMEM_SHARED((NPAD, D), jnp.float32),
        [pltpu.SemaphoreType.DMA] * 12,
    ],
)
def _scatter_kernel(src_hbm, dst_hbm, y_hbm, zrows_hbm, out_hbm,
                    src_v, dst_v, bufs, acc_s, sems):
    c = lax.axis_index("c")
    s = lax.axis_index("s")
    w = c * NS + s

    @pl.when((c == 0) & (s < NS - 1))
    def _():
        pltpu.sync_copy(y_hbm.at[pl.ds(s * RPT, RPT)],
                        acc_s.at[pl.ds(s * RPT, RPT)])

    @pl.when((c == 0) & (s == NS - 1))
    def _():
        pltpu.sync_copy(y_hbm.at[pl.ds((NS - 1) * RPT, N - (NS - 1) * RPT)],
                        acc_s.at[pl.ds((NS - 1) * RPT, N - (NS - 1) * RPT)])
        pltpu.sync_copy(zrows_hbm.at[pl.ds(0, NPAD - N)],
                        acc_s.at[pl.ds(N, NPAD - N)])

    @pl.when(c == 1)
    def _():
        pltpu.sync_copy(zrows_hbm, acc_s.at[pl.ds(s * RPT, RPT)])

    pltpu.sync_copy(src_hbm.at[w], src_v)
    pltpu.sync_copy(dst_hbm.at[w], dst_v)
    plsc.subcore_barrier()

    HF = CHS // 4
    NQ = NCHS * 4
    NU = 12

    def srcs32(q):
        return src_v.at[pl.ds(q * HF, HF)]

    def start_unit(u, q):
        pltpu.async_copy(y_hbm.at[srcs32(q)], bufs[u], sems[u])

    def wait_unit(u):
        pltpu.make_async_copy(y_hbm.at[srcs32(0)], bufs[u], sems[u]).wait()

    def scat(u, q):
        pltpu.sync_copy(bufs[u], acc_s.at[dst_v.at[pl.ds(q * HF, HF)]],
                        add=True)

    for u in range(NU):
        start_unit(u, u)

    def body(p, carry):
        for u in range(NU):
            q = NU * p + u
            wait_unit(u)
            scat(u, q)

            @pl.when(q + NU < NQ)
            def _():
                start_unit(u, q + NU)

        return carry

    lax.fori_loop(0, NQ // NU, body, 0)
    for u in range(NQ % NU):
        wait_unit(u)
        scat(u, (NQ // NU) * NU + u)
    plsc.subcore_barrier()
    pltpu.sync_copy(acc_s.at[pl.ds(s * RPT, RPT)],
                    out_hbm.at[c, pl.ds(s * RPT, RPT)])


def _matmul_body(x_ref, w_ref, xw_ref):
    xw_ref[...] = jnp.dot(x_ref[...], w_ref[...],
                          preferred_element_type=jnp.float32)


def _scale_body(xw_ref, degp_ref, y_ref, dis_ref):
    deg = degp_ref[0] + degp_ref[1] + 1.0
    dis = lax.rsqrt(deg)
    dis_ref[...] = dis
    y_ref[...] = xw_ref[...] * dis[0:N]


def _final_body(s_ref, dis_ref, b_ref, g_ref, be_ref, o_ref):
    d = dis_ref[0:N]
    h = (s_ref[0, 0:N, :] + s_ref[1, 0:N, :]) * d + b_ref[...]
    mean = jnp.mean(h, axis=0, keepdims=True)
    ctr = h - mean
    var = jnp.mean(ctr * ctr, axis=0, keepdims=True)
    o = ctr * lax.rsqrt(var + BN_EPS) * g_ref[...] + be_ref[...]
    o_ref[...] = jnp.maximum(o, 0.0)


def kernel(x, adj_t, W, b, gamma, beta):
    src = adj_t[0].astype(jnp.int32)
    dst = adj_t[1].astype(jnp.int32)
    npad = EPAD - E
    k = jnp.arange(npad, dtype=jnp.int32)
    src_p = jnp.concatenate([src, src[:npad]])
    dst_p = jnp.concatenate([dst, N + k % (NPAD - N)])
    src4 = src_p.reshape(NC * NS, NCHS * CHS)
    dst4 = dst_p.reshape(NC * NS, NCHS * CHS)
    zflat = jnp.zeros((NPAD,), jnp.float32)
    zrows = jnp.zeros((RPT, D), jnp.float32)

    xw = pl.pallas_call(
        _matmul_body,
        out_shape=jax.ShapeDtypeStruct((N, D), jnp.float32),
    )(x, W.astype(jnp.float32))

    degp = _hist_kernel(dst4, zflat)
    degp = degp.reshape(NC, NPAD, 1)

    y, dis = pl.pallas_call(
        _scale_body,
        out_shape=[jax.ShapeDtypeStruct((N, D), jnp.float32),
                   jax.ShapeDtypeStruct((NPAD, 1), jnp.float32)],
    )(xw, degp)

    sacc = _scatter_kernel(src4, dst4, y, zrows)

    out = pl.pallas_call(
        _final_body,
        out_shape=jax.ShapeDtypeStruct((N, D), jnp.float32),
    )(sacc, dis,
      b.reshape(1, D), gamma.reshape(1, D), beta.reshape(1, D))
    return out


# speedup vs baseline: 1.0878x; 1.0878x over previous
import functools

import jax
import jax.numpy as jnp
from jax import lax
from jax.experimental import pallas as pl
from jax.experimental.pallas import tpu as pltpu
from jax.experimental.pallas import tpu_sc as plsc

N = 10000
E = 320000
D = 128
NC, NS = 2, 16
CH = 128
NPAD = 10240
NCHW = 79
CHS = 64
NCHS = 158
EPAD = NC * NS * NCHW * CH
RPT = NPAD // NS
BN_EPS = 1e-5

_mesh = plsc.VectorSubcoreMesh(core_axis_name="c", subcore_axis_name="s")


@functools.partial(
    pl.kernel,
    out_type=jax.ShapeDtypeStruct((NC, NPAD), jnp.float32),
    mesh=_mesh,
    scratch_types=[
        pltpu.VMEM((NCHW * CH,), jnp.int32),
        pltpu.VMEM((CH,), jnp.float32),
        pltpu.VMEM_SHARED((NPAD,), jnp.float32),
        pltpu.SemaphoreType.DMA,
    ],
)
def _hist_kernel(dst_hbm, zeros_hbm, out_hbm, dst_v, ones_v, hist_s, hsem):
    c = lax.axis_index("c")
    s = lax.axis_index("s")
    pltpu.sync_copy(zeros_hbm.at[pl.ds(s * RPT, RPT)],
                    hist_s.at[pl.ds(s * RPT, RPT)])
    pltpu.sync_copy(dst_hbm.at[c * NS + s], dst_v)
    for r in range(CH // 16):
        ones_v[pl.ds(r * 16, 16)] = jnp.ones((16,), jnp.float32)
    plsc.subcore_barrier()

    def body(j, carry):
        pltpu.async_copy(ones_v, hist_s.at[dst_v.at[pl.ds(j * CH, CH)]],
                         hsem, add=True)

        @pl.when(j >= 8)
        def _():
            pltpu.make_async_copy(ones_v, hist_s.at[dst_v.at[pl.ds(0, CH)]],
                              hsem).wait()

        return carry

    lax.fori_loop(0, NCHW, body, 0)

    def drain(j, carry):
        pltpu.make_async_copy(ones_v, hist_s.at[dst_v.at[pl.ds(0, CH)]],
                              hsem).wait()
        return carry

    lax.fori_loop(0, 8, drain, 0)
    plsc.subcore_barrier()
    pltpu.sync_copy(hist_s.at[pl.ds(s * RPT, RPT)],
                    out_hbm.at[c].at[pl.ds(s * RPT, RPT)])


@functools.partial(
    pl.kernel,
    out_type=jax.ShapeDtypeStruct((NC, NPAD, D), jnp.float32),
    mesh=_mesh,
    scratch_types=[
        pltpu.VMEM((NCHS * CHS,), jnp.int32),
        pltpu.VMEM((NCHS * CHS,), jnp.int32),
        [pltpu.VMEM((CHS // 2, D), jnp.float32)] * 6,
        pltpu.VMEM_SHARED((NPAD, D), jnp.float32),
        [pltpu.SemaphoreType.DMA] * 6,
    ],
)
def _scatter_kernel(src_hbm, dst_hbm, y_hbm, zrows_hbm, out_hbm,
                    src_v, dst_v, bufs, acc_s, sems):
    c = lax.axis_index("c")
    s = lax.axis_index("s")
    w = c * NS + s

    @pl.when((c == 0) & (s < NS - 1))
    def _():
        pltpu.sync_copy(y_hbm.at[pl.ds(s * RPT, RPT)],
                        acc_s.at[pl.ds(s * RPT, RPT)])

    @pl.when((c == 0) & (s == NS - 1))
    def _():
        pltpu.sync_copy(y_hbm.at[pl.ds((NS - 1) * RPT, N - (NS - 1) * RPT)],
                        acc_s.at[pl.ds((NS - 1) * RPT, N - (NS - 1) * RPT)])
        pltpu.sync_copy(zrows_hbm.at[pl.ds(0, NPAD - N)],
                        acc_s.at[pl.ds(N, NPAD - N)])

    @pl.when(c == 1)
    def _():
        pltpu.sync_copy(zrows_hbm, acc_s.at[pl.ds(s * RPT, RPT)])

    pltpu.sync_copy(src_hbm.at[w], src_v)
    pltpu.sync_copy(dst_hbm.at[w], dst_v)
    plsc.subcore_barrier()

    HF = CHS // 2
    NQ = NCHS * 2
    NU = 6

    def srcs32(q):
        return src_v.at[pl.ds(q * HF, HF)]

    def start_unit(u, q):
        pltpu.async_copy(y_hbm.at[srcs32(q)], bufs[u], sems[u])

    def wait_unit(u):
        pltpu.make_async_copy(y_hbm.at[srcs32(0)], bufs[u], sems[u]).wait()

    def scat(u, q):
        pltpu.sync_copy(bufs[u], acc_s.at[dst_v.at[pl.ds(q * HF, HF)]],
                        add=True)

    for u in range(NU):
        start_unit(u, u)

    def body(p, carry):
        for u in range(NU):
            q = NU * p + u
            wait_unit(u)
            scat(u, q)

            @pl.when(q + NU < NQ)
            def _():
                start_unit(u, q + NU)

        return carry

    lax.fori_loop(0, NQ // NU, body, 0)
    for u in range(NQ % NU):
        wait_unit(u)
        scat(u, (NQ // NU) * NU + u)
    plsc.subcore_barrier()
    pltpu.sync_copy(acc_s.at[pl.ds(s * RPT, RPT)],
                    out_hbm.at[c, pl.ds(s * RPT, RPT)])


def _matmul_body(x_ref, w_ref, xw_ref):
    xw_ref[...] = jnp.dot(x_ref[...], w_ref[...],
                          preferred_element_type=jnp.float32)


def _scale_body(xw_ref, degp_ref, y_ref, dis_ref):
    deg = degp_ref[0] + degp_ref[1] + 1.0
    dis = lax.rsqrt(deg)
    dis_ref[...] = dis
    y_ref[...] = xw_ref[...] * dis[0:N]


def _final_body(s_ref, dis_ref, b_ref, g_ref, be_ref, o_ref):
    d = dis_ref[0:N]
    h = (s_ref[0, 0:N, :] + s_ref[1, 0:N, :]) * d + b_ref[...]
    mean = jnp.mean(h, axis=0, keepdims=True)
    ctr = h - mean
    var = jnp.mean(ctr * ctr, axis=0, keepdims=True)
    o = ctr * lax.rsqrt(var + BN_EPS) * g_ref[...] + be_ref[...]
    o_ref[...] = jnp.maximum(o, 0.0)


def kernel(x, adj_t, W, b, gamma, beta):
    src = adj_t[0].astype(jnp.int32)
    dst = adj_t[1].astype(jnp.int32)
    npad = EPAD - E
    k = jnp.arange(npad, dtype=jnp.int32)
    src_p = jnp.concatenate([src, src[:npad]])
    dst_p = jnp.concatenate([dst, N + k % (NPAD - N)])
    src4 = src_p.reshape(NC * NS, NCHS * CHS)
    dst4 = dst_p.reshape(NC * NS, NCHS * CHS)
    zflat = jnp.zeros((NPAD,), jnp.float32)
    zrows = jnp.zeros((RPT, D), jnp.float32)

    xw = pl.pallas_call(
        _matmul_body,
        out_shape=jax.ShapeDtypeStruct((N, D), jnp.float32),
    )(x, W.astype(jnp.float32))

    degp = _hist_kernel(dst4, zflat)
    degp = degp.reshape(NC, NPAD, 1)

    y, dis = pl.pallas_call(
        _scale_body,
        out_shape=[jax.ShapeDtypeStruct((N, D), jnp.float32),
                   jax.ShapeDtypeStruct((NPAD, 1), jnp.float32)],
    )(xw, degp)

    sacc = _scatter_kernel(src4, dst4, y, zrows)

    out = pl.pallas_call(
        _final_body,
        out_shape=jax.ShapeDtypeStruct((N, D), jnp.float32),
    )(sacc, dis,
      b.reshape(1, D), gamma.reshape(1, D), beta.reshape(1, D))
    return out
